# Initial kernel scaffold; baseline (speedup 1.0000x reference)
import jax
import jax.numpy as jnp
from jax.experimental import pallas as pl


def kernel(x, edge_index, W1, a_src1, a_dst1, b1, W2, a_src2, a_dst2, b2):
    raise NotImplementedError("write your pallas kernel here")


# speedup vs baseline: 1.7994x; 1.7994x over previous
import jax
import jax.numpy as jnp
from jax.experimental import pallas as pl

_N = 10000
_D = 128
_BLK = 1000
_SLOPE = 0.2


def _mm_body(x_ref, w_ref, asrc_ref, adst_ref, h_ref, as_ref, ad_ref):
    h = jnp.dot(x_ref[...], w_ref[...], preferred_element_type=jnp.float32)
    h_ref[...] = h
    as_ref[...] = jnp.dot(h, asrc_ref[...])
    ad_ref[...] = jnp.dot(h, adst_ref[...])


def _dense(x, W, a_s, a_d):
    return pl.pallas_call(
        _mm_body,
        out_shape=[
            jax.ShapeDtypeStruct((_N, _D), jnp.float32),
            jax.ShapeDtypeStruct((_N,), jnp.float32),
            jax.ShapeDtypeStruct((_N,), jnp.float32),
        ],
    )(x, W, a_s, a_d)


def _layer(x, src, dst, W, a_s, a_d, b):
    h, als, ald = _dense(x, W, a_s, a_d)
    e = als[src] + ald[dst]
    w = jnp.exp(jnp.where(e >= 0, e, _SLOPE * e))
    es = als + ald
    ws = jnp.exp(jnp.where(es >= 0, es, _SLOPE * es))
    denom = jax.ops.segment_sum(w, dst, num_segments=_N) + ws
    numer = jax.ops.segment_sum(h[src] * w[:, None], dst, num_segments=_N) + ws[:, None] * h
    return numer / (denom[:, None] + 1e-16) + b


def kernel(x, edge_index, W1, a_src1, a_dst1, b1, W2, a_src2, a_dst2, b2):
    src, dst = edge_index[0], edge_index[1]
    h = _layer(x, src, dst, W1, a_src1, a_dst1, b1)
    h = jax.nn.relu(h)
    out = _layer(h, src, dst, W2, a_src2, a_dst2, b2)
    return jax.nn.log_softmax(out, axis=1).astype(jnp.float32)


# speedup vs baseline: 27.5516x; 15.3118x over previous
import functools

import jax
import jax.numpy as jnp
from jax import lax
from jax.experimental import pallas as pl
from jax.experimental.pallas import tpu as pltpu
from jax.experimental.pallas import tpu_sc as plsc

_N = 10000
_D = 128
_E = 320000
_SLOPE = 0.2
_NC = 2
_NS = 16
_NW = _NC * _NS
_EB = 128
_NBLK = _E // _EB
_BPW = _NBLK // _NW
_XTRA = _NBLK - _BPW * _NW
_BLK = 2000
_CHUNK = 632
_LAST = _N - 15 * _CHUNK
_SUBS = (128, 128, 128, 128, 120)
_SUBS_LAST = (128, 128, 128, 128, 8)


def _mm_body(x_ref, w_ref, asrc_ref, adst_ref, h_ref, as_ref, ad_ref):
    h = jnp.dot(x_ref[...], w_ref[...], preferred_element_type=jnp.float32)
    h_ref[...] = h
    as_ref[...] = jnp.dot(h, asrc_ref[...], preferred_element_type=jnp.float32)
    ad_ref[...] = jnp.dot(h, adst_ref[...], preferred_element_type=jnp.float32)


def _dense(x, W, a_s, a_d):
    return pl.pallas_call(
        _mm_body,
        out_shape=[
            jax.ShapeDtypeStruct((_N, _D), jnp.float32),
            jax.ShapeDtypeStruct((_N, 1), jnp.float32),
            jax.ShapeDtypeStruct((_N, 1), jnp.float32),
        ],
    )(x, W, a_s, a_d)


def _sc_body(h_hbm, als_hbm, ald_hbm, src_hbm, dst_hbm,
             numer_hbm, den0_hbm, den1_hbm,
             as_v, ad_v, src_v, dst_v, w_v, rows_v, den_v, acc_sh, den_sh, sem):
    cid = lax.axis_index("c")
    sid = lax.axis_index("s")
    wid = cid * _NS + sid
    z16 = jnp.zeros((16,), jnp.float32)
    base = sid * _CHUNK

    pltpu.sync_copy(als_hbm, as_v)
    pltpu.sync_copy(ald_hbm, ad_v)

    def _zr(i, c):
        for j in range(8):
            rows_v[i, pl.ds(j * 16, 16)] = z16
        return c
    lax.fori_loop(0, _EB, _zr, 0)
    zrow = rows_v.at[0]

    @pl.when(sid < 15)
    def _():
        o = 0
        for ln in _SUBS:
            pltpu.sync_copy(rows_v.at[pl.ds(0, ln)],
                            acc_sh.at[pl.ds(base + o, ln)])
            pltpu.sync_copy(zrow.at[pl.ds(0, ln)],
                            den_sh.at[pl.ds(base + o, ln)])
            o += ln

    @pl.when(sid == 15)
    def _():
        o = 0
        for ln in _SUBS_LAST:
            pltpu.sync_copy(rows_v.at[pl.ds(0, ln)],
                            acc_sh.at[pl.ds(15 * _CHUNK + o, ln)])
            pltpu.sync_copy(zrow.at[pl.ds(0, ln)],
                            den_sh.at[pl.ds(15 * _CHUNK + o, ln)])
            o += ln

    plsc.subcore_barrier()

    nblk = _BPW + jnp.where(wid < _XTRA, 1, 0)
    start = wid * _BPW + jnp.minimum(wid, _XTRA)

    def _blk(bi, c):
        off = (start + bi) * _EB
        pltpu.sync_copy(src_hbm.at[pl.ds(off, _EB)], src_v)
        pltpu.sync_copy(dst_hbm.at[pl.ds(off, _EB)], dst_v)
        gat = pltpu.async_copy(h_hbm.at[src_v], rows_v, sem)
        for i in range(_EB // 16):
            i_s = src_v[pl.ds(i * 16, 16)]
            i_d = dst_v[pl.ds(i * 16, 16)]
            e = plsc.load_gather(as_v, [i_s]) + plsc.load_gather(ad_v, [i_d])
            w = jnp.exp(jnp.where(e >= 0, e, _SLOPE * e))
            w_v[pl.ds(i * 16, 16)] = w
        pltpu.sync_copy(w_v, den_sh.at[dst_v], add=True)
        gat.wait()

        def _scale(ei, cc):
            wv = plsc.load_gather(w_v, [jnp.full((16,), ei, jnp.int32)])
            for j in range(8):
                rows_v[ei, pl.ds(j * 16, 16)] = rows_v[ei, pl.ds(j * 16, 16)] * wv
            return cc
        lax.fori_loop(0, _EB, _scale, 0)
        pltpu.sync_copy(rows_v, acc_sh.at[dst_v], add=True)
        return c
    lax.fori_loop(0, nblk, _blk, 0)

    plsc.subcore_barrier()

    def _pub(subs):
        o = 0
        for ln in subs:
            pltpu.sync_copy(acc_sh.at[pl.ds(base + o, ln)],
                            rows_v.at[pl.ds(0, ln)])
            pltpu.sync_copy(rows_v.at[pl.ds(0, ln)],
                            numer_hbm.at[cid, pl.ds(base + o, ln)])
            o += ln
        pltpu.sync_copy(den_sh.at[pl.ds(base, o)], den_v.at[pl.ds(0, o)])

        @pl.when(cid == 0)
        def _():
            pltpu.sync_copy(den_v.at[pl.ds(0, o)], den0_hbm.at[pl.ds(base, o)])

        @pl.when(cid == 1)
        def _():
            pltpu.sync_copy(den_v.at[pl.ds(0, o)], den1_hbm.at[pl.ds(base, o)])

    @pl.when(sid < 15)
    def _():
        _pub(_SUBS)

    @pl.when(sid == 15)
    def _():
        _pub(_SUBS_LAST)


@functools.partial(
    pl.kernel,
    out_type=[
        jax.ShapeDtypeStruct((_NC, _N, _D), jnp.float32),
        jax.ShapeDtypeStruct((_N,), jnp.float32),
        jax.ShapeDtypeStruct((_N,), jnp.float32),
    ],
    mesh=plsc.VectorSubcoreMesh(core_axis_name="c", subcore_axis_name="s",
                                num_cores=_NC, num_subcores=_NS),
    compiler_params=pltpu.CompilerParams(needs_layout_passes=False),
    scratch_types=[
        pltpu.VMEM((_N,), jnp.float32),
        pltpu.VMEM((_N,), jnp.float32),
        pltpu.VMEM((_EB,), jnp.int32),
        pltpu.VMEM((_EB,), jnp.int32),
        pltpu.VMEM((_EB,), jnp.float32),
        pltpu.VMEM((_EB, _D), jnp.float32),
        pltpu.VMEM((_CHUNK,), jnp.float32),
        pltpu.VMEM_SHARED((_N, _D), jnp.float32),
        pltpu.VMEM_SHARED((_N,), jnp.float32),
        pltpu.SemaphoreType.DMA,
    ],
)
def _sc_edges(h_hbm, als_hbm, ald_hbm, src_hbm, dst_hbm,
              numer_hbm, den0_hbm, den1_hbm,
              as_v, ad_v, src_v, dst_v, w_v, rows_v, den_v, acc_sh, den_sh, sem):
    _sc_body(h_hbm, als_hbm, ald_hbm, src_hbm, dst_hbm,
             numer_hbm, den0_hbm, den1_hbm,
             as_v, ad_v, src_v, dst_v, w_v, rows_v, den_v, acc_sh, den_sh, sem)


def _combine(np_ref, d0_ref, d1_ref, h_ref, als_ref, ald_ref, b_ref):
    es = als_ref[...] + ald_ref[...]
    ws = jnp.exp(jnp.where(es >= 0, es, _SLOPE * es))
    numer = np_ref[0] + np_ref[1] + ws * h_ref[...]
    denom = d0_ref[...] + d1_ref[...] + ws
    return numer / (denom + 1e-16) + b_ref[...]


def _mid_body(np_ref, d0_ref, d1_ref, h_ref, als_ref, ald_ref, b_ref, w_ref,
              a2s_ref, a2d_ref, h2_ref, als2_ref, ald2_ref):
    r = jax.nn.relu(_combine(np_ref, d0_ref, d1_ref, h_ref, als_ref, ald_ref,
                             b_ref))
    h2 = jnp.dot(r, w_ref[...], preferred_element_type=jnp.float32)
    h2_ref[...] = h2
    als2_ref[...] = jnp.dot(h2, a2s_ref[...], preferred_element_type=jnp.float32)
    ald2_ref[...] = jnp.dot(h2, a2d_ref[...], preferred_element_type=jnp.float32)


def _fin_body(np_ref, d0_ref, d1_ref, h_ref, als_ref, ald_ref, b_ref, out_ref):
    o = _combine(np_ref, d0_ref, d1_ref, h_ref, als_ref, ald_ref, b_ref)
    m = jnp.max(o, axis=1, keepdims=True)
    lse = jnp.log(jnp.sum(jnp.exp(o - m), axis=1, keepdims=True)) + m
    out_ref[...] = o - lse


def _node_specs():
    return [
        pl.BlockSpec((_NC, _BLK, _D), lambda i: (0, i, 0)),
        pl.BlockSpec((_BLK, 1), lambda i: (i, 0)),
        pl.BlockSpec((_BLK, 1), lambda i: (i, 0)),
        pl.BlockSpec((_BLK, _D), lambda i: (i, 0)),
        pl.BlockSpec((_BLK, 1), lambda i: (i, 0)),
        pl.BlockSpec((_BLK, 1), lambda i: (i, 0)),
        pl.BlockSpec((_D,), lambda i: (0,)),
    ]


def _mid(np1, d0, d1, h, als, ald, b, W2, a2s, a2d):
    return pl.pallas_call(
        _mid_body,
        grid=(_N // _BLK,),
        in_specs=_node_specs() + [
            pl.BlockSpec((_D, _D), lambda i: (0, 0)),
            pl.BlockSpec((_D, 1), lambda i: (0, 0)),
            pl.BlockSpec((_D, 1), lambda i: (0, 0)),
        ],
        out_specs=[
            pl.BlockSpec((_BLK, _D), lambda i: (i, 0)),
            pl.BlockSpec((_BLK, 1), lambda i: (i, 0)),
            pl.BlockSpec((_BLK, 1), lambda i: (i, 0)),
        ],
        out_shape=[
            jax.ShapeDtypeStruct((_N, _D), jnp.float32),
            jax.ShapeDtypeStruct((_N, 1), jnp.float32),
            jax.ShapeDtypeStruct((_N, 1), jnp.float32),
        ],
    )(np1, d0, d1, h, als, ald, b, W2, a2s, a2d)


def _fin(np2, d0, d1, h, als, ald, b):
    return pl.pallas_call(
        _fin_body,
        grid=(_N // _BLK,),
        in_specs=_node_specs(),
        out_specs=pl.BlockSpec((_BLK, _D), lambda i: (i, 0)),
        out_shape=jax.ShapeDtypeStruct((_N, _D), jnp.float32),
    )(np2, d0, d1, h, als, ald, b)


def kernel(x, edge_index, W1, a_src1, a_dst1, b1, W2, a_src2, a_dst2, b2):
    src, dst = edge_index[0], edge_index[1]
    h1, als1, ald1 = _dense(x, W1, a_src1.reshape(_D, 1), a_dst1.reshape(_D, 1))
    np1, dp10, dp11 = _sc_edges(h1, als1.reshape(_N), ald1.reshape(_N), src, dst)
    h2, als2, ald2 = _mid(np1, dp10.reshape(_N, 1), dp11.reshape(_N, 1),
                          h1, als1, ald1, b1,
                          W2, a_src2.reshape(_D, 1), a_dst2.reshape(_D, 1))
    np2, dp20, dp21 = _sc_edges(h2, als2.reshape(_N), ald2.reshape(_N), src, dst)
    return _fin(np2, dp20.reshape(_N, 1), dp21.reshape(_N, 1),
                h2, als2, ald2, b2)


# speedup vs baseline: 52.1427x; 1.8925x over previous
import functools

import jax
import jax.numpy as jnp
from jax import lax
from jax.experimental import pallas as pl
from jax.experimental.pallas import tpu as pltpu
from jax.experimental.pallas import tpu_sc as plsc

_N = 10000
_D = 128
_E = 320000
_SLOPE = 0.2
_NC = 2
_NS = 16
_NW = _NC * _NS
_EB = 128
_EPW = _E // _NW
_BPW = 79
_TAILV = _EPW - (_BPW - 1) * _EB
_BLK = 2000
_CHUNK = 632
_LAST = _N - 15 * _CHUNK
_SUBS = (128, 128, 128, 128, 120)
_SUBS_LAST = (128, 128, 128, 128, 8)


def _mm_body(x_ref, w_ref, asrc_ref, adst_ref, h_ref, as_ref, ad_ref):
    h = jnp.dot(x_ref[...], w_ref[...], preferred_element_type=jnp.float32)
    h_ref[...] = h
    as_ref[...] = jnp.dot(h, asrc_ref[...], preferred_element_type=jnp.float32)
    ad_ref[...] = jnp.dot(h, adst_ref[...], preferred_element_type=jnp.float32)


def _dense(x, W, a_s, a_d):
    return pl.pallas_call(
        _mm_body,
        out_shape=[
            jax.ShapeDtypeStruct((_N, _D), jnp.float32),
            jax.ShapeDtypeStruct((_N, 1), jnp.float32),
            jax.ShapeDtypeStruct((_N, 1), jnp.float32),
        ],
    )(x, W, a_s, a_d)


def _scale(rv, wv):
    def body(ei, c):
        e0 = ei * 2
        w0 = plsc.load_gather(wv, [jnp.full((16,), e0, jnp.int32)])
        w1 = plsc.load_gather(wv, [jnp.full((16,), e0 + 1, jnp.int32)])
        for j in range(8):
            rv[e0, pl.ds(j * 16, 16)] = rv[e0, pl.ds(j * 16, 16)] * w0
            rv[e0 + 1, pl.ds(j * 16, 16)] = rv[e0 + 1, pl.ds(j * 16, 16)] * w1
        return c
    lax.fori_loop(0, _EB // 2, body, 0)


def _sc_body(h_hbm, als_hbm, ald_hbm, src_hbm, dst_hbm,
             numer_hbm, den0_hbm, den1_hbm,
             den_v, src_v, dst_v, w_v, asv, adv, rows, acc_sh, den_sh,
             isrc_sem, idst_sem, asv_sem, adv_sem, gat_sem):
    cid = lax.axis_index("c")
    sid = lax.axis_index("s")
    wid = cid * _NS + sid
    z16 = jnp.zeros((16,), jnp.float32)
    base = sid * _CHUNK
    ebase = wid * _EPW

    def _zr(i, c):
        for j in range(8):
            rows[0][i, pl.ds(j * 16, 16)] = z16
        return c
    lax.fori_loop(0, _EB, _zr, 0)
    zrow = rows[0].at[0]

    def _zero(subs):
        o = 0
        for ln in subs:
            pltpu.sync_copy(rows[0].at[pl.ds(0, ln)],
                            acc_sh.at[pl.ds(base + o, ln)])
            pltpu.sync_copy(zrow.at[pl.ds(0, ln)],
                            den_sh.at[pl.ds(base + o, ln)])
            o += ln

    @pl.when(sid < 15)
    def _():
        _zero(_SUBS)

    @pl.when(sid == 15)
    def _():
        _zero(_SUBS_LAST)

    plsc.subcore_barrier()

    def _idx_fetch(b, slot):
        off = ebase + b * _EB
        pltpu.async_copy(src_hbm.at[pl.ds(off, _EB)], src_v[slot],
                         isrc_sem[slot])
        pltpu.async_copy(dst_hbm.at[pl.ds(off, _EB)], dst_v[slot],
                         idst_sem[slot])

    def _idx_wait(slot):
        pltpu.make_async_copy(src_hbm.at[pl.ds(0, _EB)], src_v[slot],
                              isrc_sem[slot]).wait()
        pltpu.make_async_copy(src_hbm.at[pl.ds(0, _EB)], dst_v[slot],
                              idst_sem[slot]).wait()

    def _issue_gathers(i3, i2):
        pltpu.async_copy(h_hbm.at[src_v[i3]], rows[i2], gat_sem[i2])
        pltpu.async_copy(als_hbm.at[src_v[i3]], asv[i3], asv_sem[i3])
        pltpu.async_copy(ald_hbm.at[dst_v[i3]], adv[i3], adv_sem[i3])

    def _proc(k3, k2, tail):
        pltpu.make_async_copy(als_hbm.at[src_v[k3]], asv[k3],
                              asv_sem[k3]).wait()
        pltpu.make_async_copy(ald_hbm.at[dst_v[k3]], adv[k3],
                              adv_sem[k3]).wait()
        for i in range(_EB // 16):
            e = asv[k3][pl.ds(i * 16, 16)] + adv[k3][pl.ds(i * 16, 16)]
            w_v[k3][pl.ds(i * 16, 16)] = jnp.exp(
                jnp.where(e >= 0, e, _SLOPE * e))
        if tail:
            for t in range(_TAILV // 16, _EB // 16):
                w_v[k3][pl.ds(t * 16, 16)] = z16
        pltpu.sync_copy(w_v[k3], den_sh.at[dst_v[k3]], add=True)
        pltpu.make_async_copy(h_hbm.at[src_v[k3]], rows[k2],
                              gat_sem[k2]).wait()
        _scale(rows[k2], w_v[k3])
        pltpu.sync_copy(rows[k2], acc_sh.at[dst_v[k3]], add=True)

    _idx_fetch(0, 0)
    _idx_fetch(1, 1)
    _idx_wait(0)
    _issue_gathers(0, 0)
    nmain = _BPW - 1

    def _step(it, c):
        for k6 in range(6):
            b = it * 6 + k6
            k3 = k6 % 3
            k2 = k6 % 2

            @pl.when(b + 2 < nmain)
            def _():
                _idx_fetch(b + 2, (k3 + 2) % 3)

            @pl.when(b + 1 < nmain)
            def _():
                _idx_wait((k3 + 1) % 3)
                _issue_gathers((k3 + 1) % 3, (k2 + 1) % 2)

            _proc(k3, k2, tail=False)
        return c
    lax.fori_loop(0, nmain // 6, _step, 0)

    _idx_fetch(nmain, 0)
    _idx_wait(0)
    _issue_gathers(0, 0)
    _proc(0, 0, tail=True)

    plsc.subcore_barrier()

    def _pub(subs):
        o = 0
        for ln in subs:
            pltpu.sync_copy(acc_sh.at[pl.ds(base + o, ln)],
                            rows[0].at[pl.ds(0, ln)])
            pltpu.sync_copy(rows[0].at[pl.ds(0, ln)],
                            numer_hbm.at[cid, pl.ds(base + o, ln)])
            o += ln
        pltpu.sync_copy(den_sh.at[pl.ds(base, o)], den_v.at[pl.ds(0, o)])

        @pl.when(cid == 0)
        def _():
            pltpu.sync_copy(den_v.at[pl.ds(0, o)], den0_hbm.at[pl.ds(base, o)])

        @pl.when(cid == 1)
        def _():
            pltpu.sync_copy(den_v.at[pl.ds(0, o)], den1_hbm.at[pl.ds(base, o)])

    @pl.when(sid < 15)
    def _():
        _pub(_SUBS)

    @pl.when(sid == 15)
    def _():
        _pub(_SUBS_LAST)


_SC_SCRATCH = (
    [pltpu.VMEM((_CHUNK,), jnp.float32)]
    + [pltpu.VMEM((_EB,), jnp.int32)] * 6
    + [pltpu.VMEM((_EB,), jnp.float32)] * 3
    + [pltpu.VMEM((_EB,), jnp.float32)] * 6
    + [pltpu.VMEM((_EB, _D), jnp.float32)] * 2
    + [pltpu.VMEM_SHARED((_N, _D), jnp.float32),
       pltpu.VMEM_SHARED((_N,), jnp.float32)]
    + [pltpu.SemaphoreType.DMA] * 14
)


@functools.partial(
    pl.kernel,
    out_type=[
        jax.ShapeDtypeStruct((_NC, _N, _D), jnp.float32),
        jax.ShapeDtypeStruct((_N,), jnp.float32),
        jax.ShapeDtypeStruct((_N,), jnp.float32),
    ],
    mesh=plsc.VectorSubcoreMesh(core_axis_name="c", subcore_axis_name="s",
                                num_cores=_NC, num_subcores=_NS),
    compiler_params=pltpu.CompilerParams(needs_layout_passes=False),
    scratch_types=_SC_SCRATCH,
)
def _sc_edges(h_hbm, als_hbm, ald_hbm, src_hbm, dst_hbm,
              numer_hbm, den0_hbm, den1_hbm, *scr):
    den_v = scr[0]
    src_v = scr[1:4]
    dst_v = scr[4:7]
    w_v = scr[7:10]
    asv = scr[10:13]
    adv = scr[13:16]
    rows = scr[16:18]
    acc_sh, den_sh = scr[18], scr[19]
    isrc_sem = scr[20:23]
    idst_sem = scr[23:26]
    asv_sem = scr[26:29]
    adv_sem = scr[29:32]
    gat_sem = scr[32:34]
    _sc_body(h_hbm, als_hbm, ald_hbm, src_hbm, dst_hbm,
             numer_hbm, den0_hbm, den1_hbm,
             den_v, src_v, dst_v, w_v, asv, adv, rows, acc_sh, den_sh,
             isrc_sem, idst_sem, asv_sem, adv_sem, gat_sem)


def _combine(np_ref, d0_ref, d1_ref, h_ref, als_ref, ald_ref, b_ref):
    es = als_ref[...] + ald_ref[...]
    ws = jnp.exp(jnp.where(es >= 0, es, _SLOPE * es))
    numer = np_ref[0] + np_ref[1] + ws * h_ref[...]
    denom = d0_ref[...] + d1_ref[...] + ws
    return numer / (denom + 1e-16) + b_ref[...]


def _mid_body(np_ref, d0_ref, d1_ref, h_ref, als_ref, ald_ref, b_ref, w_ref,
              a2s_ref, a2d_ref, h2_ref, als2_ref, ald2_ref):
    r = jax.nn.relu(_combine(np_ref, d0_ref, d1_ref, h_ref, als_ref, ald_ref,
                             b_ref))
    h2 = jnp.dot(r, w_ref[...], preferred_element_type=jnp.float32)
    h2_ref[...] = h2
    als2_ref[...] = jnp.dot(h2, a2s_ref[...], preferred_element_type=jnp.float32)
    ald2_ref[...] = jnp.dot(h2, a2d_ref[...], preferred_element_type=jnp.float32)


def _fin_body(np_ref, d0_ref, d1_ref, h_ref, als_ref, ald_ref, b_ref, out_ref):
    o = _combine(np_ref, d0_ref, d1_ref, h_ref, als_ref, ald_ref, b_ref)
    m = jnp.max(o, axis=1, keepdims=True)
    lse = jnp.log(jnp.sum(jnp.exp(o - m), axis=1, keepdims=True)) + m
    out_ref[...] = o - lse


def _node_specs():
    return [
        pl.BlockSpec((_NC, _BLK, _D), lambda i: (0, i, 0)),
        pl.BlockSpec((_BLK, 1), lambda i: (i, 0)),
        pl.BlockSpec((_BLK, 1), lambda i: (i, 0)),
        pl.BlockSpec((_BLK, _D), lambda i: (i, 0)),
        pl.BlockSpec((_BLK, 1), lambda i: (i, 0)),
        pl.BlockSpec((_BLK, 1), lambda i: (i, 0)),
        pl.BlockSpec((_D,), lambda i: (0,)),
    ]


def _mid(np1, d0, d1, h, als, ald, b, W2, a2s, a2d):
    return pl.pallas_call(
        _mid_body,
        grid=(_N // _BLK,),
        in_specs=_node_specs() + [
            pl.BlockSpec((_D, _D), lambda i: (0, 0)),
            pl.BlockSpec((_D, 1), lambda i: (0, 0)),
            pl.BlockSpec((_D, 1), lambda i: (0, 0)),
        ],
        out_specs=[
            pl.BlockSpec((_BLK, _D), lambda i: (i, 0)),
            pl.BlockSpec((_BLK, 1), lambda i: (i, 0)),
            pl.BlockSpec((_BLK, 1), lambda i: (i, 0)),
        ],
        out_shape=[
            jax.ShapeDtypeStruct((_N, _D), jnp.float32),
            jax.ShapeDtypeStruct((_N, 1), jnp.float32),
            jax.ShapeDtypeStruct((_N, 1), jnp.float32),
        ],
    )(np1, d0, d1, h, als, ald, b, W2, a2s, a2d)


def _fin(np2, d0, d1, h, als, ald, b):
    return pl.pallas_call(
        _fin_body,
        grid=(_N // _BLK,),
        in_specs=_node_specs(),
        out_specs=pl.BlockSpec((_BLK, _D), lambda i: (i, 0)),
        out_shape=jax.ShapeDtypeStruct((_N, _D), jnp.float32),
    )(np2, d0, d1, h, als, ald, b)


def kernel(x, edge_index, W1, a_src1, a_dst1, b1, W2, a_src2, a_dst2, b2):
    pad = jnp.zeros((_BPW * _EB - _EPW,), jnp.int32)
    src = jnp.concatenate([edge_index[0], pad])
    dst = jnp.concatenate([edge_index[1], pad])
    h1, als1, ald1 = _dense(x, W1, a_src1.reshape(_D, 1), a_dst1.reshape(_D, 1))
    np1, dp10, dp11 = _sc_edges(h1, als1.reshape(_N), ald1.reshape(_N), src, dst)
    h2, als2, ald2 = _mid(np1, dp10.reshape(_N, 1), dp11.reshape(_N, 1),
                          h1, als1, ald1, b1,
                          W2, a_src2.reshape(_D, 1), a_dst2.reshape(_D, 1))
    np2, dp20, dp21 = _sc_edges(h2, als2.reshape(_N), ald2.reshape(_N), src, dst)
    return _fin(np2, dp20.reshape(_N, 1), dp21.reshape(_N, 1),
                h2, als2, ald2, b2)


# speedup vs baseline: 54.0547x; 1.0367x over previous
import functools

import jax
import jax.numpy as jnp
from jax import lax
from jax.experimental import pallas as pl
from jax.experimental.pallas import tpu as pltpu
from jax.experimental.pallas import tpu_sc as plsc

_N = 10000
_D = 128
_E = 320000
_SLOPE = 0.2
_NC = 2
_NS = 16
_NW = _NC * _NS
_EB = 128
_EPW = _E // _NW
_BPW = 79
_TAILV = _EPW - (_BPW - 1) * _EB
_BLK = 2000
_CHUNK = 632
_LAST = _N - 15 * _CHUNK
_SUBS = (128, 128, 128, 128, 120)
_SUBS_LAST = (128, 128, 128, 128, 8)


def _mm_body(x_ref, w_ref, asrc_ref, adst_ref, h_ref, as_ref, ad_ref):
    h = jnp.dot(x_ref[...], w_ref[...], preferred_element_type=jnp.float32)
    h_ref[...] = h
    as_ref[...] = jnp.dot(h, asrc_ref[...], preferred_element_type=jnp.float32)
    ad_ref[...] = jnp.dot(h, adst_ref[...], preferred_element_type=jnp.float32)


def _dense(x, W, a_s, a_d):
    return pl.pallas_call(
        _mm_body,
        out_shape=[
            jax.ShapeDtypeStruct((_N, _D), jnp.float32),
            jax.ShapeDtypeStruct((_N, 1), jnp.float32),
            jax.ShapeDtypeStruct((_N, 1), jnp.float32),
        ],
    )(x, W, a_s, a_d)


def _scale(rv, wv):
    def body(ei, c):
        e0 = ei * 2
        w0 = plsc.load_gather(wv, [jnp.full((16,), e0, jnp.int32)])
        w1 = plsc.load_gather(wv, [jnp.full((16,), e0 + 1, jnp.int32)])
        for j in range(8):
            rv[e0, pl.ds(j * 16, 16)] = rv[e0, pl.ds(j * 16, 16)] * w0
            rv[e0 + 1, pl.ds(j * 16, 16)] = rv[e0 + 1, pl.ds(j * 16, 16)] * w1
        return c
    lax.fori_loop(0, _EB // 2, body, 0)


def _sc_body(h_hbm, als_hbm, ald_hbm, src_hbm, dst_hbm,
             numer_hbm, den0_hbm, den1_hbm,
             den_v, src_v, dst_v, w_v, asv, adv, rows, acc_sh, den_sh,
             isrc_sem, idst_sem, asv_sem, adv_sem, gat_sem, den_sem,
             scat_sem):
    cid = lax.axis_index("c")
    sid = lax.axis_index("s")
    wid = cid * _NS + sid
    z16 = jnp.zeros((16,), jnp.float32)
    base = sid * _CHUNK
    ebase = wid * _EPW

    def _zr(i, c):
        for j in range(8):
            rows[0][i, pl.ds(j * 16, 16)] = z16
        return c
    lax.fori_loop(0, _EB, _zr, 0)
    zrow = rows[0].at[0]

    def _zero(subs):
        o = 0
        for ln in subs:
            pltpu.sync_copy(rows[0].at[pl.ds(0, ln)],
                            acc_sh.at[pl.ds(base + o, ln)])
            pltpu.sync_copy(zrow.at[pl.ds(0, ln)],
                            den_sh.at[pl.ds(base + o, ln)])
            o += ln

    @pl.when(sid < 15)
    def _():
        _zero(_SUBS)

    @pl.when(sid == 15)
    def _():
        _zero(_SUBS_LAST)

    plsc.subcore_barrier()

    def _idx_fetch(b, s6, guard):
        if guard:
            @pl.when(b >= 6)
            def _():
                pltpu.make_async_copy(w_v[s6], den_sh.at[dst_v[s6]],
                                      den_sem[s6]).wait()
        off = ebase + b * _EB
        pltpu.async_copy(src_hbm.at[pl.ds(off, _EB)], src_v[s6],
                         isrc_sem[s6])
        pltpu.async_copy(dst_hbm.at[pl.ds(off, _EB)], dst_v[s6],
                         idst_sem[s6])

    def _idx_wait(s6):
        pltpu.make_async_copy(src_hbm.at[pl.ds(0, _EB)], src_v[s6],
                              isrc_sem[s6]).wait()
        pltpu.make_async_copy(src_hbm.at[pl.ds(0, _EB)], dst_v[s6],
                              idst_sem[s6]).wait()

    def _issue_logit_gathers(s6, s3):
        pltpu.async_copy(als_hbm.at[src_v[s6]], asv[s3], asv_sem[s3])
        pltpu.async_copy(ald_hbm.at[dst_v[s6]], adv[s3], adv_sem[s3])

    def _issue_row_gather(b, s6, s2, guard):
        if guard:
            @pl.when(b >= 2)
            def _():
                pltpu.make_async_copy(rows[s2], acc_sh.at[dst_v[s6]],
                                      scat_sem[s2]).wait()
        pltpu.async_copy(h_hbm.at[src_v[s6]], rows[s2], gat_sem[s2])

    def _weights(s6, s3, tail, sync_scatter):
        pltpu.make_async_copy(als_hbm.at[src_v[s6]], asv[s3],
                              asv_sem[s3]).wait()
        pltpu.make_async_copy(ald_hbm.at[dst_v[s6]], adv[s3],
                              adv_sem[s3]).wait()
        for i in range(_EB // 16):
            e = asv[s3][pl.ds(i * 16, 16)] + adv[s3][pl.ds(i * 16, 16)]
            w_v[s6][pl.ds(i * 16, 16)] = jnp.exp(
                jnp.where(e >= 0, e, _SLOPE * e))
        if tail:
            for t in range(_TAILV // 16, _EB // 16):
                w_v[s6][pl.ds(t * 16, 16)] = z16
        if sync_scatter:
            pltpu.sync_copy(w_v[s6], den_sh.at[dst_v[s6]], add=True)
        else:
            pltpu.async_copy(w_v[s6], den_sh.at[dst_v[s6]], den_sem[s6],
                             add=True)

    def _rows(s6, s2, sync_scatter):
        pltpu.make_async_copy(h_hbm.at[src_v[s6]], rows[s2],
                              gat_sem[s2]).wait()
        _scale(rows[s2], w_v[s6])
        if sync_scatter:
            pltpu.sync_copy(rows[s2], acc_sh.at[dst_v[s6]], add=True)
        else:
            pltpu.async_copy(rows[s2], acc_sh.at[dst_v[s6]], scat_sem[s2],
                             add=True)

    _idx_fetch(0, 0, guard=False)
    _idx_fetch(1, 1, guard=False)
    _idx_wait(0)
    _issue_logit_gathers(0, 0)
    _issue_row_gather(0, 0, 0, guard=False)
    nmain = _BPW - 1

    def _step(it, c):
        for k6 in range(6):
            b = it * 6 + k6
            k3 = k6 % 3
            k2 = k6 % 2

            @pl.when(b + 2 < nmain)
            def _():
                _idx_fetch(b + 2, (k6 + 2) % 6, guard=True)

            @pl.when(b + 1 < nmain)
            def _():
                _idx_wait((k6 + 1) % 6)
                _issue_logit_gathers((k6 + 1) % 6, (k3 + 1) % 3)

            _weights(k6, k3, tail=False, sync_scatter=False)

            @pl.when(b + 1 < nmain)
            def _():
                _issue_row_gather(b + 1, (k6 + 1) % 6, (k2 + 1) % 2,
                                  guard=True)

            _rows(k6, k2, sync_scatter=False)
        return c
    lax.fori_loop(0, nmain // 6, _step, 0)

    for s in range(6):
        pltpu.make_async_copy(w_v[s], den_sh.at[dst_v[s]], den_sem[s]).wait()
    for s in range(2):
        pltpu.make_async_copy(rows[s], acc_sh.at[dst_v[s]],
                              scat_sem[s]).wait()

    _idx_fetch(nmain, 0, guard=False)
    _idx_wait(0)
    _issue_logit_gathers(0, 0)
    _issue_row_gather(nmain, 0, 0, guard=False)
    _weights(0, 0, tail=True, sync_scatter=True)
    _rows(0, 0, sync_scatter=True)

    plsc.subcore_barrier()

    def _pub(subs):
        o = 0
        for ln in subs:
            pltpu.sync_copy(acc_sh.at[pl.ds(base + o, ln)],
                            rows[0].at[pl.ds(0, ln)])
            pltpu.sync_copy(rows[0].at[pl.ds(0, ln)],
                            numer_hbm.at[cid, pl.ds(base + o, ln)])
            o += ln
        pltpu.sync_copy(den_sh.at[pl.ds(base, o)], den_v.at[pl.ds(0, o)])

        @pl.when(cid == 0)
        def _():
            pltpu.sync_copy(den_v.at[pl.ds(0, o)], den0_hbm.at[pl.ds(base, o)])

        @pl.when(cid == 1)
        def _():
            pltpu.sync_copy(den_v.at[pl.ds(0, o)], den1_hbm.at[pl.ds(base, o)])

    @pl.when(sid < 15)
    def _():
        _pub(_SUBS)

    @pl.when(sid == 15)
    def _():
        _pub(_SUBS_LAST)


_SC_SCRATCH = (
    [pltpu.VMEM((_CHUNK,), jnp.float32)]
    + [pltpu.VMEM((_EB,), jnp.int32)] * 12
    + [pltpu.VMEM((_EB,), jnp.float32)] * 6
    + [pltpu.VMEM((_EB,), jnp.float32)] * 6
    + [pltpu.VMEM((_EB, _D), jnp.float32)] * 2
    + [pltpu.VMEM_SHARED((_N, _D), jnp.float32),
       pltpu.VMEM_SHARED((_N,), jnp.float32)]
    + [pltpu.SemaphoreType.DMA] * 28
)


@functools.partial(
    pl.kernel,
    out_type=[
        jax.ShapeDtypeStruct((_NC, _N, _D), jnp.float32),
        jax.ShapeDtypeStruct((_N,), jnp.float32),
        jax.ShapeDtypeStruct((_N,), jnp.float32),
    ],
    mesh=plsc.VectorSubcoreMesh(core_axis_name="c", subcore_axis_name="s",
                                num_cores=_NC, num_subcores=_NS),
    compiler_params=pltpu.CompilerParams(needs_layout_passes=False),
    scratch_types=_SC_SCRATCH,
)
def _sc_edges(h_hbm, als_hbm, ald_hbm, src_hbm, dst_hbm,
              numer_hbm, den0_hbm, den1_hbm, *scr):
    den_v = scr[0]
    src_v = scr[1:7]
    dst_v = scr[7:13]
    w_v = scr[13:19]
    asv = scr[19:22]
    adv = scr[22:25]
    rows = scr[25:27]
    acc_sh, den_sh = scr[27], scr[28]
    isrc_sem = scr[29:35]
    idst_sem = scr[35:41]
    asv_sem = scr[41:44]
    adv_sem = scr[44:47]
    gat_sem = scr[47:49]
    den_sem = scr[49:55]
    scat_sem = scr[55:57]
    _sc_body(h_hbm, als_hbm, ald_hbm, src_hbm, dst_hbm,
             numer_hbm, den0_hbm, den1_hbm,
             den_v, src_v, dst_v, w_v, asv, adv, rows, acc_sh, den_sh,
             isrc_sem, idst_sem, asv_sem, adv_sem, gat_sem, den_sem,
             scat_sem)


def _combine(np_ref, d0_ref, d1_ref, h_ref, als_ref, ald_ref, b_ref):
    es = als_ref[...] + ald_ref[...]
    ws = jnp.exp(jnp.where(es >= 0, es, _SLOPE * es))
    numer = np_ref[0] + np_ref[1] + ws * h_ref[...]
    denom = d0_ref[...] + d1_ref[...] + ws
    return numer / (denom + 1e-16) + b_ref[...]


def _mid_body(np_ref, d0_ref, d1_ref, h_ref, als_ref, ald_ref, b_ref, w_ref,
              a2s_ref, a2d_ref, h2_ref, als2_ref, ald2_ref):
    r = jax.nn.relu(_combine(np_ref, d0_ref, d1_ref, h_ref, als_ref, ald_ref,
                             b_ref))
    h2 = jnp.dot(r, w_ref[...], preferred_element_type=jnp.float32)
    h2_ref[...] = h2
    als2_ref[...] = jnp.dot(h2, a2s_ref[...], preferred_element_type=jnp.float32)
    ald2_ref[...] = jnp.dot(h2, a2d_ref[...], preferred_element_type=jnp.float32)


def _fin_body(np_ref, d0_ref, d1_ref, h_ref, als_ref, ald_ref, b_ref, out_ref):
    o = _combine(np_ref, d0_ref, d1_ref, h_ref, als_ref, ald_ref, b_ref)
    m = jnp.max(o, axis=1, keepdims=True)
    lse = jnp.log(jnp.sum(jnp.exp(o - m), axis=1, keepdims=True)) + m
    out_ref[...] = o - lse


def _node_specs():
    return [
        pl.BlockSpec((_NC, _BLK, _D), lambda i: (0, i, 0)),
        pl.BlockSpec((_BLK, 1), lambda i: (i, 0)),
        pl.BlockSpec((_BLK, 1), lambda i: (i, 0)),
        pl.BlockSpec((_BLK, _D), lambda i: (i, 0)),
        pl.BlockSpec((_BLK, 1), lambda i: (i, 0)),
        pl.BlockSpec((_BLK, 1), lambda i: (i, 0)),
        pl.BlockSpec((_D,), lambda i: (0,)),
    ]


def _mid(np1, d0, d1, h, als, ald, b, W2, a2s, a2d):
    return pl.pallas_call(
        _mid_body,
        grid=(_N // _BLK,),
        in_specs=_node_specs() + [
            pl.BlockSpec((_D, _D), lambda i: (0, 0)),
            pl.BlockSpec((_D, 1), lambda i: (0, 0)),
            pl.BlockSpec((_D, 1), lambda i: (0, 0)),
        ],
        out_specs=[
            pl.BlockSpec((_BLK, _D), lambda i: (i, 0)),
            pl.BlockSpec((_BLK, 1), lambda i: (i, 0)),
            pl.BlockSpec((_BLK, 1), lambda i: (i, 0)),
        ],
        out_shape=[
            jax.ShapeDtypeStruct((_N, _D), jnp.float32),
            jax.ShapeDtypeStruct((_N, 1), jnp.float32),
            jax.ShapeDtypeStruct((_N, 1), jnp.float32),
        ],
    )(np1, d0, d1, h, als, ald, b, W2, a2s, a2d)


def _fin(np2, d0, d1, h, als, ald, b):
    return pl.pallas_call(
        _fin_body,
        grid=(_N // _BLK,),
        in_specs=_node_specs(),
        out_specs=pl.BlockSpec((_BLK, _D), lambda i: (i, 0)),
        out_shape=jax.ShapeDtypeStruct((_N, _D), jnp.float32),
    )(np2, d0, d1, h, als, ald, b)


def kernel(x, edge_index, W1, a_src1, a_dst1, b1, W2, a_src2, a_dst2, b2):
    pad = jnp.zeros((_BPW * _EB - _EPW,), jnp.int32)
    src = jnp.concatenate([edge_index[0], pad])
    dst = jnp.concatenate([edge_index[1], pad])
    h1, als1, ald1 = _dense(x, W1, a_src1.reshape(_D, 1), a_dst1.reshape(_D, 1))
    np1, dp10, dp11 = _sc_edges(h1, als1.reshape(_N), ald1.reshape(_N), src, dst)
    h2, als2, ald2 = _mid(np1, dp10.reshape(_N, 1), dp11.reshape(_N, 1),
                          h1, als1, ald1, b1,
                          W2, a_src2.reshape(_D, 1), a_dst2.reshape(_D, 1))
    np2, dp20, dp21 = _sc_edges(h2, als2.reshape(_N), ald2.reshape(_N), src, dst)
    return _fin(np2, dp20.reshape(_N, 1), dp21.reshape(_N, 1),
                h2, als2, ald2, b2)


# speedup vs baseline: 56.2632x; 1.0409x over previous
import functools

import jax
import jax.numpy as jnp
from jax import lax
from jax.experimental import pallas as pl
from jax.experimental.pallas import tpu as pltpu
from jax.experimental.pallas import tpu_sc as plsc

_N = 10000
_D = 128
_E = 320000
_SLOPE = 0.2
_NC = 2
_NS = 16
_NW = _NC * _NS
_EB = 128
_EPW = _E // _NW
_BPW = 79
_TAILV = _EPW - (_BPW - 1) * _EB
_BLK = 2000
_CHUNK = 632
_LAST = _N - 15 * _CHUNK
_SUBS = (128, 128, 128, 128, 120)
_SUBS_LAST = (128, 128, 128, 128, 8)


def _mm_body(x_ref, w_ref, asrc_ref, adst_ref, h_ref, as_ref, ad_ref):
    h = jnp.dot(x_ref[...], w_ref[...], preferred_element_type=jnp.float32)
    h_ref[...] = h
    as_ref[...] = jnp.dot(h, asrc_ref[...], preferred_element_type=jnp.float32)
    ad_ref[...] = jnp.dot(h, adst_ref[...], preferred_element_type=jnp.float32)


def _dense(x, W, a_s, a_d):
    return pl.pallas_call(
        _mm_body,
        out_shape=[
            jax.ShapeDtypeStruct((_N, _D), jnp.float32),
            jax.ShapeDtypeStruct((_N, 1), jnp.float32),
            jax.ShapeDtypeStruct((_N, 1), jnp.float32),
        ],
    )(x, W, a_s, a_d)


def _scale(rv, wv):
    def body(ei, c):
        e0 = ei * 4
        ws = [plsc.load_gather(wv, [jnp.full((16,), e0 + k, jnp.int32)])
              for k in range(4)]
        for k in range(4):
            for j in range(8):
                rv[e0 + k, pl.ds(j * 16, 16)] = (
                    rv[e0 + k, pl.ds(j * 16, 16)] * ws[k])
        return c
    lax.fori_loop(0, _EB // 4, body, 0)


def _sc_body(h_hbm, als_hbm, ald_hbm, src_hbm, dst_hbm,
             numer_hbm, den0_hbm, den1_hbm,
             den_v, src_v, dst_v, w_v, asv, adv, rows, acc_sh, den_sh,
             isrc_sem, idst_sem, asv_sem, adv_sem, gat_sem, den_sem,
             scat_sem):
    cid = lax.axis_index("c")
    sid = lax.axis_index("s")
    wid = cid * _NS + sid
    z16 = jnp.zeros((16,), jnp.float32)
    base = sid * _CHUNK
    ebase = wid * _EPW

    def _zr(i, c):
        for j in range(8):
            rows[0][i, pl.ds(j * 16, 16)] = z16
        return c
    lax.fori_loop(0, _EB, _zr, 0)
    zrow = rows[0].at[0]

    def _zero(subs):
        o = 0
        for ln in subs:
            pltpu.sync_copy(rows[0].at[pl.ds(0, ln)],
                            acc_sh.at[pl.ds(base + o, ln)])
            pltpu.sync_copy(zrow.at[pl.ds(0, ln)],
                            den_sh.at[pl.ds(base + o, ln)])
            o += ln

    @pl.when(sid < 15)
    def _():
        _zero(_SUBS)

    @pl.when(sid == 15)
    def _():
        _zero(_SUBS_LAST)

    plsc.subcore_barrier()

    def _idx_fetch(b, s6, guard):
        if guard:
            @pl.when(b >= 6)
            def _():
                pltpu.make_async_copy(w_v[s6], den_sh.at[dst_v[s6]],
                                      den_sem[s6]).wait()
        off = ebase + b * _EB
        pltpu.async_copy(src_hbm.at[pl.ds(off, _EB)], src_v[s6],
                         isrc_sem[s6])
        pltpu.async_copy(dst_hbm.at[pl.ds(off, _EB)], dst_v[s6],
                         idst_sem[s6])

    def _idx_wait(s6):
        pltpu.make_async_copy(src_hbm.at[pl.ds(0, _EB)], src_v[s6],
                              isrc_sem[s6]).wait()
        pltpu.make_async_copy(src_hbm.at[pl.ds(0, _EB)], dst_v[s6],
                              idst_sem[s6]).wait()

    def _issue_logit_gathers(s6, s3):
        pltpu.async_copy(als_hbm.at[src_v[s6]], asv[s3], asv_sem[s3])
        pltpu.async_copy(ald_hbm.at[dst_v[s6]], adv[s3], adv_sem[s3])

    def _issue_row_gather(b, s6, s2, guard):
        if guard:
            @pl.when(b >= 2)
            def _():
                pltpu.make_async_copy(rows[s2], acc_sh.at[dst_v[s6]],
                                      scat_sem[s2]).wait()
        pltpu.async_copy(h_hbm.at[src_v[s6]], rows[s2], gat_sem[s2])

    def _weights(s6, s3, tail, sync_scatter):
        pltpu.make_async_copy(als_hbm.at[src_v[s6]], asv[s3],
                              asv_sem[s3]).wait()
        pltpu.make_async_copy(ald_hbm.at[dst_v[s6]], adv[s3],
                              adv_sem[s3]).wait()
        for i in range(_EB // 16):
            e = asv[s3][pl.ds(i * 16, 16)] + adv[s3][pl.ds(i * 16, 16)]
            w_v[s6][pl.ds(i * 16, 16)] = jnp.exp(
                jnp.where(e >= 0, e, _SLOPE * e))
        if tail:
            for t in range(_TAILV // 16, _EB // 16):
                w_v[s6][pl.ds(t * 16, 16)] = z16
        if sync_scatter:
            pltpu.sync_copy(w_v[s6], den_sh.at[dst_v[s6]], add=True)
        else:
            pltpu.async_copy(w_v[s6], den_sh.at[dst_v[s6]], den_sem[s6],
                             add=True)

    def _rows(s6, s2, sync_scatter):
        pltpu.make_async_copy(h_hbm.at[src_v[s6]], rows[s2],
                              gat_sem[s2]).wait()
        _scale(rows[s2], w_v[s6])
        if sync_scatter:
            pltpu.sync_copy(rows[s2], acc_sh.at[dst_v[s6]], add=True)
        else:
            pltpu.async_copy(rows[s2], acc_sh.at[dst_v[s6]], scat_sem[s2],
                             add=True)

    _idx_fetch(0, 0, guard=False)
    _idx_fetch(1, 1, guard=False)
    _idx_wait(0)
    _issue_logit_gathers(0, 0)
    _issue_row_gather(0, 0, 0, guard=False)
    nmain = _BPW - 1

    def _step(it, c):
        for k6 in range(6):
            b = it * 6 + k6
            k3 = k6 % 3
            k2 = k6 % 2

            @pl.when(b + 2 < nmain)
            def _():
                _idx_fetch(b + 2, (k6 + 2) % 6, guard=True)

            @pl.when(b + 1 < nmain)
            def _():
                _idx_wait((k6 + 1) % 6)
                _issue_logit_gathers((k6 + 1) % 6, (k3 + 1) % 3)

            _weights(k6, k3, tail=False, sync_scatter=False)

            @pl.when(b + 1 < nmain)
            def _():
                _issue_row_gather(b + 1, (k6 + 1) % 6, (k2 + 1) % 2,
                                  guard=True)

            _rows(k6, k2, sync_scatter=False)
        return c
    lax.fori_loop(0, nmain // 6, _step, 0)

    for s in range(6):
        pltpu.make_async_copy(w_v[s], den_sh.at[dst_v[s]], den_sem[s]).wait()
    for s in range(2):
        pltpu.make_async_copy(rows[s], acc_sh.at[dst_v[s]],
                              scat_sem[s]).wait()

    _idx_fetch(nmain, 0, guard=False)
    _idx_wait(0)
    _issue_logit_gathers(0, 0)
    _issue_row_gather(nmain, 0, 0, guard=False)
    _weights(0, 0, tail=True, sync_scatter=True)
    _rows(0, 0, sync_scatter=True)

    plsc.subcore_barrier()

    def _pub(subs):
        o = 0
        for ln in subs:
            pltpu.sync_copy(acc_sh.at[pl.ds(base + o, ln)],
                            rows[0].at[pl.ds(0, ln)])
            pltpu.sync_copy(rows[0].at[pl.ds(0, ln)],
                            numer_hbm.at[cid, pl.ds(base + o, ln)])
            o += ln
        pltpu.sync_copy(den_sh.at[pl.ds(base, o)], den_v.at[pl.ds(0, o)])

        @pl.when(cid == 0)
        def _():
            pltpu.sync_copy(den_v.at[pl.ds(0, o)], den0_hbm.at[pl.ds(base, o)])

        @pl.when(cid == 1)
        def _():
            pltpu.sync_copy(den_v.at[pl.ds(0, o)], den1_hbm.at[pl.ds(base, o)])

    @pl.when(sid < 15)
    def _():
        _pub(_SUBS)

    @pl.when(sid == 15)
    def _():
        _pub(_SUBS_LAST)


_SC_SCRATCH = (
    [pltpu.VMEM((_CHUNK,), jnp.float32)]
    + [pltpu.VMEM((_EB,), jnp.int32)] * 12
    + [pltpu.VMEM((_EB,), jnp.float32)] * 6
    + [pltpu.VMEM((_EB,), jnp.float32)] * 6
    + [pltpu.VMEM((_EB, _D), jnp.float32)] * 2
    + [pltpu.VMEM_SHARED((_N, _D), jnp.float32),
       pltpu.VMEM_SHARED((_N,), jnp.float32)]
    + [pltpu.SemaphoreType.DMA] * 28
)


@functools.partial(
    pl.kernel,
    out_type=[
        jax.ShapeDtypeStruct((_NC, _N, _D), jnp.float32),
        jax.ShapeDtypeStruct((_N,), jnp.float32),
        jax.ShapeDtypeStruct((_N,), jnp.float32),
    ],
    mesh=plsc.VectorSubcoreMesh(core_axis_name="c", subcore_axis_name="s",
                                num_cores=_NC, num_subcores=_NS),
    compiler_params=pltpu.CompilerParams(needs_layout_passes=False),
    scratch_types=_SC_SCRATCH,
)
def _sc_edges(h_hbm, als_hbm, ald_hbm, src_hbm, dst_hbm,
              numer_hbm, den0_hbm, den1_hbm, *scr):
    den_v = scr[0]
    src_v = scr[1:7]
    dst_v = scr[7:13]
    w_v = scr[13:19]
    asv = scr[19:22]
    adv = scr[22:25]
    rows = scr[25:27]
    acc_sh, den_sh = scr[27], scr[28]
    isrc_sem = scr[29:35]
    idst_sem = scr[35:41]
    asv_sem = scr[41:44]
    adv_sem = scr[44:47]
    gat_sem = scr[47:49]
    den_sem = scr[49:55]
    scat_sem = scr[55:57]
    _sc_body(h_hbm, als_hbm, ald_hbm, src_hbm, dst_hbm,
             numer_hbm, den0_hbm, den1_hbm,
             den_v, src_v, dst_v, w_v, asv, adv, rows, acc_sh, den_sh,
             isrc_sem, idst_sem, asv_sem, adv_sem, gat_sem, den_sem,
             scat_sem)


def _combine(np_ref, d0_ref, d1_ref, h_ref, als_ref, ald_ref, b_ref):
    es = als_ref[...] + ald_ref[...]
    ws = jnp.exp(jnp.where(es >= 0, es, _SLOPE * es))
    numer = np_ref[0] + np_ref[1] + ws * h_ref[...]
    denom = d0_ref[...] + d1_ref[...] + ws
    return numer / (denom + 1e-16) + b_ref[...]


def _mid_body(np_ref, d0_ref, d1_ref, h_ref, als_ref, ald_ref, b_ref, w_ref,
              a2s_ref, a2d_ref, h2_ref, als2_ref, ald2_ref):
    r = jax.nn.relu(_combine(np_ref, d0_ref, d1_ref, h_ref, als_ref, ald_ref,
                             b_ref))
    h2 = jnp.dot(r, w_ref[...], preferred_element_type=jnp.float32)
    h2_ref[...] = h2
    als2_ref[...] = jnp.dot(h2, a2s_ref[...], preferred_element_type=jnp.float32)
    ald2_ref[...] = jnp.dot(h2, a2d_ref[...], preferred_element_type=jnp.float32)


def _fin_body(np_ref, d0_ref, d1_ref, h_ref, als_ref, ald_ref, b_ref, out_ref):
    o = _combine(np_ref, d0_ref, d1_ref, h_ref, als_ref, ald_ref, b_ref)
    m = jnp.max(o, axis=1, keepdims=True)
    lse = jnp.log(jnp.sum(jnp.exp(o - m), axis=1, keepdims=True)) + m
    out_ref[...] = o - lse


def _node_specs():
    return [
        pl.BlockSpec((_NC, _BLK, _D), lambda i: (0, i, 0)),
        pl.BlockSpec((_BLK, 1), lambda i: (i, 0)),
        pl.BlockSpec((_BLK, 1), lambda i: (i, 0)),
        pl.BlockSpec((_BLK, _D), lambda i: (i, 0)),
        pl.BlockSpec((_BLK, 1), lambda i: (i, 0)),
        pl.BlockSpec((_BLK, 1), lambda i: (i, 0)),
        pl.BlockSpec((_D,), lambda i: (0,)),
    ]


def _mid(np1, d0, d1, h, als, ald, b, W2, a2s, a2d):
    return pl.pallas_call(
        _mid_body,
        grid=(_N // _BLK,),
        in_specs=_node_specs() + [
            pl.BlockSpec((_D, _D), lambda i: (0, 0)),
            pl.BlockSpec((_D, 1), lambda i: (0, 0)),
            pl.BlockSpec((_D, 1), lambda i: (0, 0)),
        ],
        out_specs=[
            pl.BlockSpec((_BLK, _D), lambda i: (i, 0)),
            pl.BlockSpec((_BLK, 1), lambda i: (i, 0)),
            pl.BlockSpec((_BLK, 1), lambda i: (i, 0)),
        ],
        out_shape=[
            jax.ShapeDtypeStruct((_N, _D), jnp.float32),
            jax.ShapeDtypeStruct((_N, 1), jnp.float32),
            jax.ShapeDtypeStruct((_N, 1), jnp.float32),
        ],
    )(np1, d0, d1, h, als, ald, b, W2, a2s, a2d)


def _fin(np2, d0, d1, h, als, ald, b):
    return pl.pallas_call(
        _fin_body,
        grid=(_N // _BLK,),
        in_specs=_node_specs(),
        out_specs=pl.BlockSpec((_BLK, _D), lambda i: (i, 0)),
        out_shape=jax.ShapeDtypeStruct((_N, _D), jnp.float32),
    )(np2, d0, d1, h, als, ald, b)


def kernel(x, edge_index, W1, a_src1, a_dst1, b1, W2, a_src2, a_dst2, b2):
    pad = jnp.zeros((_BPW * _EB - _EPW,), jnp.int32)
    src = jnp.concatenate([edge_index[0], pad])
    dst = jnp.concatenate([edge_index[1], pad])
    h1, als1, ald1 = _dense(x, W1, a_src1.reshape(_D, 1), a_dst1.reshape(_D, 1))
    np1, dp10, dp11 = _sc_edges(h1, als1.reshape(_N), ald1.reshape(_N), src, dst)
    h2, als2, ald2 = _mid(np1, dp10.reshape(_N, 1), dp11.reshape(_N, 1),
                          h1, als1, ald1, b1,
                          W2, a_src2.reshape(_D, 1), a_dst2.reshape(_D, 1))
    np2, dp20, dp21 = _sc_edges(h2, als2.reshape(_N), ald2.reshape(_N), src, dst)
    return _fin(np2, dp20.reshape(_N, 1), dp21.reshape(_N, 1),
                h2, als2, ald2, b2)


# speedup vs baseline: 56.6081x; 1.0061x over previous
import functools

import jax
import jax.numpy as jnp
from jax import lax
from jax.experimental import pallas as pl
from jax.experimental.pallas import tpu as pltpu
from jax.experimental.pallas import tpu_sc as plsc

_N = 10000
_D = 128
_E = 320000
_SLOPE = 0.2
_NC = 2
_NS = 16
_NW = _NC * _NS
_EB = 128
_EPW = _E // _NW
_BPW = 79
_TAILV = _EPW - (_BPW - 1) * _EB
_BLK = 2000
_CHUNK = 632
_LAST = _N - 15 * _CHUNK
_SUBS = (128, 128, 128, 128, 120)
_SUBS_LAST = (128, 128, 128, 128, 8)


def _mm_body(x_ref, w_ref, asrc_ref, adst_ref, h_ref, as_ref, ad_ref):
    h = jnp.dot(x_ref[...], w_ref[...], preferred_element_type=jnp.float32)
    h_ref[...] = h
    as_ref[...] = jnp.dot(h, asrc_ref[...], preferred_element_type=jnp.float32)
    ad_ref[...] = jnp.dot(h, adst_ref[...], preferred_element_type=jnp.float32)


def _dense(x, W, a_s, a_d):
    return pl.pallas_call(
        _mm_body,
        out_shape=[
            jax.ShapeDtypeStruct((_N, _D), jnp.float32),
            jax.ShapeDtypeStruct((_N, 1), jnp.float32),
            jax.ShapeDtypeStruct((_N, 1), jnp.float32),
        ],
    )(x, W, a_s, a_d)


def _scale(rv, wv):
    def body(ei, c):
        e0 = ei * 8
        ws = [plsc.load_gather(wv, [jnp.full((16,), e0 + k, jnp.int32)])
              for k in range(8)]
        for k in range(8):
            for j in range(8):
                rv[e0 + k, pl.ds(j * 16, 16)] = (
                    rv[e0 + k, pl.ds(j * 16, 16)] * ws[k])
        return c
    lax.fori_loop(0, _EB // 8, body, 0)


def _sc_body(h_hbm, als_hbm, ald_hbm, src_hbm, dst_hbm,
             numer_hbm, den0_hbm, den1_hbm,
             den_v, src_v, dst_v, w_v, asv, adv, rows, acc_sh, den_sh,
             isrc_sem, idst_sem, asv_sem, adv_sem, gat_sem, den_sem,
             scat_sem):
    cid = lax.axis_index("c")
    sid = lax.axis_index("s")
    wid = cid * _NS + sid
    z16 = jnp.zeros((16,), jnp.float32)
    base = sid * _CHUNK
    ebase = wid * _EPW

    def _zr(i, c):
        for j in range(8):
            rows[0][i, pl.ds(j * 16, 16)] = z16
        return c
    lax.fori_loop(0, _EB, _zr, 0)
    zrow = rows[0].at[0]

    def _zero(subs):
        o = 0
        for ln in subs:
            pltpu.sync_copy(rows[0].at[pl.ds(0, ln)],
                            acc_sh.at[pl.ds(base + o, ln)])
            pltpu.sync_copy(zrow.at[pl.ds(0, ln)],
                            den_sh.at[pl.ds(base + o, ln)])
            o += ln

    @pl.when(sid < 15)
    def _():
        _zero(_SUBS)

    @pl.when(sid == 15)
    def _():
        _zero(_SUBS_LAST)

    plsc.subcore_barrier()

    def _idx_fetch(b, s6, guard):
        if guard:
            @pl.when(b >= 6)
            def _():
                pltpu.make_async_copy(w_v[s6], den_sh.at[dst_v[s6]],
                                      den_sem[s6]).wait()
        off = ebase + b * _EB
        pltpu.async_copy(src_hbm.at[pl.ds(off, _EB)], src_v[s6],
                         isrc_sem[s6])
        pltpu.async_copy(dst_hbm.at[pl.ds(off, _EB)], dst_v[s6],
                         idst_sem[s6])

    def _idx_wait(s6):
        pltpu.make_async_copy(src_hbm.at[pl.ds(0, _EB)], src_v[s6],
                              isrc_sem[s6]).wait()
        pltpu.make_async_copy(src_hbm.at[pl.ds(0, _EB)], dst_v[s6],
                              idst_sem[s6]).wait()

    def _issue_logit_gathers(s6, s3):
        pltpu.async_copy(als_hbm.at[src_v[s6]], asv[s3], asv_sem[s3])
        pltpu.async_copy(ald_hbm.at[dst_v[s6]], adv[s3], adv_sem[s3])

    def _issue_row_gather(b, s6, s2, guard):
        if guard:
            @pl.when(b >= 2)
            def _():
                pltpu.make_async_copy(rows[s2], acc_sh.at[dst_v[s6]],
                                      scat_sem[s2]).wait()
        pltpu.async_copy(h_hbm.at[src_v[s6]], rows[s2], gat_sem[s2])

    def _weights(s6, s3, tail, sync_scatter):
        pltpu.make_async_copy(als_hbm.at[src_v[s6]], asv[s3],
                              asv_sem[s3]).wait()
        pltpu.make_async_copy(ald_hbm.at[dst_v[s6]], adv[s3],
                              adv_sem[s3]).wait()
        for i in range(_EB // 16):
            e = asv[s3][pl.ds(i * 16, 16)] + adv[s3][pl.ds(i * 16, 16)]
            w_v[s6][pl.ds(i * 16, 16)] = jnp.exp(
                jnp.where(e >= 0, e, _SLOPE * e))
        if tail:
            for t in range(_TAILV // 16, _EB // 16):
                w_v[s6][pl.ds(t * 16, 16)] = z16
        if sync_scatter:
            pltpu.sync_copy(w_v[s6], den_sh.at[dst_v[s6]], add=True)
        else:
            pltpu.async_copy(w_v[s6], den_sh.at[dst_v[s6]], den_sem[s6],
                             add=True)

    def _rows(s6, s2, sync_scatter):
        pltpu.make_async_copy(h_hbm.at[src_v[s6]], rows[s2],
                              gat_sem[s2]).wait()
        _scale(rows[s2], w_v[s6])
        if sync_scatter:
            pltpu.sync_copy(rows[s2], acc_sh.at[dst_v[s6]], add=True)
        else:
            pltpu.async_copy(rows[s2], acc_sh.at[dst_v[s6]], scat_sem[s2],
                             add=True)

    _idx_fetch(0, 0, guard=False)
    _idx_fetch(1, 1, guard=False)
    _idx_wait(0)
    _issue_logit_gathers(0, 0)
    _issue_row_gather(0, 0, 0, guard=False)
    nmain = _BPW - 1

    def _step(it, c):
        for k6 in range(6):
            b = it * 6 + k6
            k3 = k6 % 3
            k2 = k6 % 2

            @pl.when(b + 2 < nmain)
            def _():
                _idx_fetch(b + 2, (k6 + 2) % 6, guard=True)

            @pl.when(b + 1 < nmain)
            def _():
                _idx_wait((k6 + 1) % 6)
                _issue_logit_gathers((k6 + 1) % 6, (k3 + 1) % 3)

            _weights(k6, k3, tail=False, sync_scatter=False)

            @pl.when(b + 1 < nmain)
            def _():
                _issue_row_gather(b + 1, (k6 + 1) % 6, (k2 + 1) % 2,
                                  guard=True)

            _rows(k6, k2, sync_scatter=False)
        return c
    lax.fori_loop(0, nmain // 6, _step, 0)

    for s in range(6):
        pltpu.make_async_copy(w_v[s], den_sh.at[dst_v[s]], den_sem[s]).wait()
    for s in range(2):
        pltpu.make_async_copy(rows[s], acc_sh.at[dst_v[s]],
                              scat_sem[s]).wait()

    _idx_fetch(nmain, 0, guard=False)
    _idx_wait(0)
    _issue_logit_gathers(0, 0)
    _issue_row_gather(nmain, 0, 0, guard=False)
    _weights(0, 0, tail=True, sync_scatter=True)
    _rows(0, 0, sync_scatter=True)

    plsc.subcore_barrier()

    def _pub(subs):
        o = 0
        for ln in subs:
            pltpu.sync_copy(acc_sh.at[pl.ds(base + o, ln)],
                            rows[0].at[pl.ds(0, ln)])
            pltpu.sync_copy(rows[0].at[pl.ds(0, ln)],
                            numer_hbm.at[cid, pl.ds(base + o, ln)])
            o += ln
        pltpu.sync_copy(den_sh.at[pl.ds(base, o)], den_v.at[pl.ds(0, o)])

        @pl.when(cid == 0)
        def _():
            pltpu.sync_copy(den_v.at[pl.ds(0, o)], den0_hbm.at[pl.ds(base, o)])

        @pl.when(cid == 1)
        def _():
            pltpu.sync_copy(den_v.at[pl.ds(0, o)], den1_hbm.at[pl.ds(base, o)])

    @pl.when(sid < 15)
    def _():
        _pub(_SUBS)

    @pl.when(sid == 15)
    def _():
        _pub(_SUBS_LAST)


_SC_SCRATCH = (
    [pltpu.VMEM((_CHUNK,), jnp.float32)]
    + [pltpu.VMEM((_EB,), jnp.int32)] * 12
    + [pltpu.VMEM((_EB,), jnp.float32)] * 6
    + [pltpu.VMEM((_EB,), jnp.float32)] * 6
    + [pltpu.VMEM((_EB, _D), jnp.float32)] * 2
    + [pltpu.VMEM_SHARED((_N, _D), jnp.float32),
       pltpu.VMEM_SHARED((_N,), jnp.float32)]
    + [pltpu.SemaphoreType.DMA] * 28
)


@functools.partial(
    pl.kernel,
    out_type=[
        jax.ShapeDtypeStruct((_NC, _N, _D), jnp.float32),
        jax.ShapeDtypeStruct((_N,), jnp.float32),
        jax.ShapeDtypeStruct((_N,), jnp.float32),
    ],
    mesh=plsc.VectorSubcoreMesh(core_axis_name="c", subcore_axis_name="s",
                                num_cores=_NC, num_subcores=_NS),
    compiler_params=pltpu.CompilerParams(needs_layout_passes=False),
    scratch_types=_SC_SCRATCH,
)
def _sc_edges(h_hbm, als_hbm, ald_hbm, src_hbm, dst_hbm,
              numer_hbm, den0_hbm, den1_hbm, *scr):
    den_v = scr[0]
    src_v = scr[1:7]
    dst_v = scr[7:13]
    w_v = scr[13:19]
    asv = scr[19:22]
    adv = scr[22:25]
    rows = scr[25:27]
    acc_sh, den_sh = scr[27], scr[28]
    isrc_sem = scr[29:35]
    idst_sem = scr[35:41]
    asv_sem = scr[41:44]
    adv_sem = scr[44:47]
    gat_sem = scr[47:49]
    den_sem = scr[49:55]
    scat_sem = scr[55:57]
    _sc_body(h_hbm, als_hbm, ald_hbm, src_hbm, dst_hbm,
             numer_hbm, den0_hbm, den1_hbm,
             den_v, src_v, dst_v, w_v, asv, adv, rows, acc_sh, den_sh,
             isrc_sem, idst_sem, asv_sem, adv_sem, gat_sem, den_sem,
             scat_sem)


def _combine(np_ref, d0_ref, d1_ref, h_ref, als_ref, ald_ref, b_ref):
    es = als_ref[...] + ald_ref[...]
    ws = jnp.exp(jnp.where(es >= 0, es, _SLOPE * es))
    numer = np_ref[0] + np_ref[1] + ws * h_ref[...]
    denom = d0_ref[...] + d1_ref[...] + ws
    return numer / (denom + 1e-16) + b_ref[...]


def _mid_body(np_ref, d0_ref, d1_ref, h_ref, als_ref, ald_ref, b_ref, w_ref,
              a2s_ref, a2d_ref, h2_ref, als2_ref, ald2_ref):
    r = jax.nn.relu(_combine(np_ref, d0_ref, d1_ref, h_ref, als_ref, ald_ref,
                             b_ref))
    h2 = jnp.dot(r, w_ref[...], preferred_element_type=jnp.float32)
    h2_ref[...] = h2
    als2_ref[...] = jnp.dot(h2, a2s_ref[...], preferred_element_type=jnp.float32)
    ald2_ref[...] = jnp.dot(h2, a2d_ref[...], preferred_element_type=jnp.float32)


def _fin_body(np_ref, d0_ref, d1_ref, h_ref, als_ref, ald_ref, b_ref, out_ref):
    o = _combine(np_ref, d0_ref, d1_ref, h_ref, als_ref, ald_ref, b_ref)
    m = jnp.max(o, axis=1, keepdims=True)
    lse = jnp.log(jnp.sum(jnp.exp(o - m), axis=1, keepdims=True)) + m
    out_ref[...] = o - lse


def _node_specs():
    return [
        pl.BlockSpec((_NC, _BLK, _D), lambda i: (0, i, 0)),
        pl.BlockSpec((_BLK, 1), lambda i: (i, 0)),
        pl.BlockSpec((_BLK, 1), lambda i: (i, 0)),
        pl.BlockSpec((_BLK, _D), lambda i: (i, 0)),
        pl.BlockSpec((_BLK, 1), lambda i: (i, 0)),
        pl.BlockSpec((_BLK, 1), lambda i: (i, 0)),
        pl.BlockSpec((_D,), lambda i: (0,)),
    ]


def _mid(np1, d0, d1, h, als, ald, b, W2, a2s, a2d):
    return pl.pallas_call(
        _mid_body,
        grid=(_N // _BLK,),
        in_specs=_node_specs() + [
            pl.BlockSpec((_D, _D), lambda i: (0, 0)),
            pl.BlockSpec((_D, 1), lambda i: (0, 0)),
            pl.BlockSpec((_D, 1), lambda i: (0, 0)),
        ],
        out_specs=[
            pl.BlockSpec((_BLK, _D), lambda i: (i, 0)),
            pl.BlockSpec((_BLK, 1), lambda i: (i, 0)),
            pl.BlockSpec((_BLK, 1), lambda i: (i, 0)),
        ],
        out_shape=[
            jax.ShapeDtypeStruct((_N, _D), jnp.float32),
            jax.ShapeDtypeStruct((_N, 1), jnp.float32),
            jax.ShapeDtypeStruct((_N, 1), jnp.float32),
        ],
    )(np1, d0, d1, h, als, ald, b, W2, a2s, a2d)


def _fin(np2, d0, d1, h, als, ald, b):
    return pl.pallas_call(
        _fin_body,
        grid=(_N // _BLK,),
        in_specs=_node_specs(),
        out_specs=pl.BlockSpec((_BLK, _D), lambda i: (i, 0)),
        out_shape=jax.ShapeDtypeStruct((_N, _D), jnp.float32),
    )(np2, d0, d1, h, als, ald, b)


def kernel(x, edge_index, W1, a_src1, a_dst1, b1, W2, a_src2, a_dst2, b2):
    pad = jnp.zeros((_BPW * _EB - _EPW,), jnp.int32)
    src = jnp.concatenate([edge_index[0], pad])
    dst = jnp.concatenate([edge_index[1], pad])
    h1, als1, ald1 = _dense(x, W1, a_src1.reshape(_D, 1), a_dst1.reshape(_D, 1))
    np1, dp10, dp11 = _sc_edges(h1, als1.reshape(_N), ald1.reshape(_N), src, dst)
    h2, als2, ald2 = _mid(np1, dp10.reshape(_N, 1), dp11.reshape(_N, 1),
                          h1, als1, ald1, b1,
                          W2, a_src2.reshape(_D, 1), a_dst2.reshape(_D, 1))
    np2, dp20, dp21 = _sc_edges(h2, als2.reshape(_N), ald2.reshape(_N), src, dst)
    return _fin(np2, dp20.reshape(_N, 1), dp21.reshape(_N, 1),
                h2, als2, ald2, b2)


# speedup vs baseline: 63.0324x; 1.1135x over previous
import functools

import jax
import jax.numpy as jnp
from jax import lax
from jax.experimental import pallas as pl
from jax.experimental.pallas import tpu as pltpu
from jax.experimental.pallas import tpu_sc as plsc

_N = 10000
_D = 128
_E = 320000
_SLOPE = 0.2
_NC = 2
_NS = 16
_NW = _NC * _NS
_EB = 128
_EPW = _E // _NW
_BPW = 79
_TAILV = _EPW - (_BPW - 1) * _EB
_BLK = 2000
_CHUNK = 632
_LAST = _N - 15 * _CHUNK
_SUBS = (128, 128, 128, 128, 120)
_SUBS_LAST = (128, 128, 128, 128, 8)


def _mm_body(x_ref, w_ref, asrc_ref, adst_ref, h_ref, as_ref, ad_ref):
    h = jnp.dot(x_ref[...], w_ref[...], preferred_element_type=jnp.float32)
    h_ref[...] = h
    as_ref[...] = jnp.dot(h, asrc_ref[...],
                          preferred_element_type=jnp.float32)[:, 0]
    ad_ref[...] = jnp.dot(h, adst_ref[...],
                          preferred_element_type=jnp.float32)[:, 0]


def _dense(x, W, a_s, a_d):
    return pl.pallas_call(
        _mm_body,
        out_shape=[
            jax.ShapeDtypeStruct((_N, _D), jnp.float32),
            jax.ShapeDtypeStruct((_N,), jnp.float32),
            jax.ShapeDtypeStruct((_N,), jnp.float32),
        ],
    )(x, W, a_s, a_d)


def _scale(rv, wv):
    def body(ei, c):
        e0 = ei * 8
        ws = [plsc.load_gather(wv, [jnp.full((16,), e0 + k, jnp.int32)])
              for k in range(8)]
        for k in range(8):
            for j in range(8):
                rv[e0 + k, pl.ds(j * 16, 16)] = (
                    rv[e0 + k, pl.ds(j * 16, 16)] * ws[k])
        return c
    lax.fori_loop(0, _EB // 8, body, 0)


def _sc_body(h_hbm, als_hbm, ald_hbm, src_hbm, dst_hbm,
             numer_hbm, den0_hbm, den1_hbm,
             den_v, src_v, dst_v, w_v, asv, adv, rows, acc_sh, den_sh,
             isrc_sem, idst_sem, asv_sem, adv_sem, gat_sem, den_sem,
             scat_sem):
    cid = lax.axis_index("c")
    sid = lax.axis_index("s")
    wid = cid * _NS + sid
    z16 = jnp.zeros((16,), jnp.float32)
    base = sid * _CHUNK
    ebase = wid * _EPW

    def _zr(i, c):
        for j in range(8):
            rows[0][i, pl.ds(j * 16, 16)] = z16
        return c
    lax.fori_loop(0, _EB, _zr, 0)
    zrow = rows[0].at[0]

    def _zero(subs):
        o = 0
        for ln in subs:
            pltpu.sync_copy(rows[0].at[pl.ds(0, ln)],
                            acc_sh.at[pl.ds(base + o, ln)])
            pltpu.sync_copy(zrow.at[pl.ds(0, ln)],
                            den_sh.at[pl.ds(base + o, ln)])
            o += ln

    @pl.when(sid < 15)
    def _():
        _zero(_SUBS)

    @pl.when(sid == 15)
    def _():
        _zero(_SUBS_LAST)

    plsc.subcore_barrier()

    def _idx_fetch(b, s6, guard):
        if guard:
            @pl.when(b >= 6)
            def _():
                pltpu.make_async_copy(w_v[s6], den_sh.at[dst_v[s6]],
                                      den_sem[s6]).wait()
        off = ebase + b * _EB
        pltpu.async_copy(src_hbm.at[pl.ds(off, _EB)], src_v[s6],
                         isrc_sem[s6])
        pltpu.async_copy(dst_hbm.at[pl.ds(off, _EB)], dst_v[s6],
                         idst_sem[s6])

    def _idx_wait(s6):
        pltpu.make_async_copy(src_hbm.at[pl.ds(0, _EB)], src_v[s6],
                              isrc_sem[s6]).wait()
        pltpu.make_async_copy(src_hbm.at[pl.ds(0, _EB)], dst_v[s6],
                              idst_sem[s6]).wait()

    def _issue_logit_gathers(s6, s3):
        pltpu.async_copy(als_hbm.at[src_v[s6]], asv[s3], asv_sem[s3])
        pltpu.async_copy(ald_hbm.at[dst_v[s6]], adv[s3], adv_sem[s3])

    def _issue_row_gather(b, s6, s2, guard):
        if guard:
            @pl.when(b >= 2)
            def _():
                pltpu.make_async_copy(rows[s2], acc_sh.at[dst_v[s6]],
                                      scat_sem[s2]).wait()
        pltpu.async_copy(h_hbm.at[src_v[s6]], rows[s2], gat_sem[s2])

    def _weights(s6, s3, tail, sync_scatter):
        pltpu.make_async_copy(als_hbm.at[src_v[s6]], asv[s3],
                              asv_sem[s3]).wait()
        pltpu.make_async_copy(ald_hbm.at[dst_v[s6]], adv[s3],
                              adv_sem[s3]).wait()
        for i in range(_EB // 16):
            e = asv[s3][pl.ds(i * 16, 16)] + adv[s3][pl.ds(i * 16, 16)]
            w_v[s6][pl.ds(i * 16, 16)] = jnp.exp(
                jnp.where(e >= 0, e, _SLOPE * e))
        if tail:
            for t in range(_TAILV // 16, _EB // 16):
                w_v[s6][pl.ds(t * 16, 16)] = z16
        if sync_scatter:
            pltpu.sync_copy(w_v[s6], den_sh.at[dst_v[s6]], add=True)
        else:
            pltpu.async_copy(w_v[s6], den_sh.at[dst_v[s6]], den_sem[s6],
                             add=True)

    def _rows(s6, s2, sync_scatter):
        pltpu.make_async_copy(h_hbm.at[src_v[s6]], rows[s2],
                              gat_sem[s2]).wait()
        _scale(rows[s2], w_v[s6])
        if sync_scatter:
            pltpu.sync_copy(rows[s2], acc_sh.at[dst_v[s6]], add=True)
        else:
            pltpu.async_copy(rows[s2], acc_sh.at[dst_v[s6]], scat_sem[s2],
                             add=True)

    _idx_fetch(0, 0, guard=False)
    _idx_fetch(1, 1, guard=False)
    _idx_wait(0)
    _issue_logit_gathers(0, 0)
    _issue_row_gather(0, 0, 0, guard=False)
    nmain = _BPW - 1

    def _step(it, c):
        for k6 in range(6):
            b = it * 6 + k6
            k3 = k6 % 3
            k2 = k6 % 2

            @pl.when(b + 2 < nmain)
            def _():
                _idx_fetch(b + 2, (k6 + 2) % 6, guard=True)

            @pl.when(b + 1 < nmain)
            def _():
                _idx_wait((k6 + 1) % 6)
                _issue_logit_gathers((k6 + 1) % 6, (k3 + 1) % 3)

            _weights(k6, k3, tail=False, sync_scatter=False)

            @pl.when(b + 1 < nmain)
            def _():
                _issue_row_gather(b + 1, (k6 + 1) % 6, (k2 + 1) % 2,
                                  guard=True)

            _rows(k6, k2, sync_scatter=False)
        return c
    lax.fori_loop(0, nmain // 6, _step, 0)

    for s in range(6):
        pltpu.make_async_copy(w_v[s], den_sh.at[dst_v[s]], den_sem[s]).wait()
    for s in range(2):
        pltpu.make_async_copy(rows[s], acc_sh.at[dst_v[s]],
                              scat_sem[s]).wait()

    _idx_fetch(nmain, 0, guard=False)
    _idx_wait(0)
    _issue_logit_gathers(0, 0)
    _issue_row_gather(nmain, 0, 0, guard=False)
    _weights(0, 0, tail=True, sync_scatter=True)
    _rows(0, 0, sync_scatter=True)

    plsc.subcore_barrier()

    def _pub(subs):
        o = 0
        for ln in subs:
            pltpu.sync_copy(acc_sh.at[pl.ds(base + o, ln)],
                            rows[0].at[pl.ds(0, ln)])
            pltpu.sync_copy(rows[0].at[pl.ds(0, ln)],
                            numer_hbm.at[cid, pl.ds(base + o, ln)])
            o += ln
        pltpu.sync_copy(den_sh.at[pl.ds(base, o)], den_v.at[pl.ds(0, o)])

        @pl.when(cid == 0)
        def _():
            pltpu.sync_copy(den_v.at[pl.ds(0, o)], den0_hbm.at[pl.ds(base, o)])

        @pl.when(cid == 1)
        def _():
            pltpu.sync_copy(den_v.at[pl.ds(0, o)], den1_hbm.at[pl.ds(base, o)])

    @pl.when(sid < 15)
    def _():
        _pub(_SUBS)

    @pl.when(sid == 15)
    def _():
        _pub(_SUBS_LAST)


_SC_SCRATCH = (
    [pltpu.VMEM((_CHUNK,), jnp.float32)]
    + [pltpu.VMEM((_EB,), jnp.int32)] * 12
    + [pltpu.VMEM((_EB,), jnp.float32)] * 6
    + [pltpu.VMEM((_EB,), jnp.float32)] * 6
    + [pltpu.VMEM((_EB, _D), jnp.float32)] * 2
    + [pltpu.VMEM_SHARED((_N, _D), jnp.float32),
       pltpu.VMEM_SHARED((_N,), jnp.float32)]
    + [pltpu.SemaphoreType.DMA] * 28
)


@functools.partial(
    pl.kernel,
    out_type=[
        jax.ShapeDtypeStruct((_NC, _N, _D), jnp.float32),
        jax.ShapeDtypeStruct((_N,), jnp.float32),
        jax.ShapeDtypeStruct((_N,), jnp.float32),
    ],
    mesh=plsc.VectorSubcoreMesh(core_axis_name="c", subcore_axis_name="s",
                                num_cores=_NC, num_subcores=_NS),
    compiler_params=pltpu.CompilerParams(needs_layout_passes=False),
    scratch_types=_SC_SCRATCH,
)
def _sc_edges(h_hbm, als_hbm, ald_hbm, src_hbm, dst_hbm,
              numer_hbm, den0_hbm, den1_hbm, *scr):
    den_v = scr[0]
    src_v = scr[1:7]
    dst_v = scr[7:13]
    w_v = scr[13:19]
    asv = scr[19:22]
    adv = scr[22:25]
    rows = scr[25:27]
    acc_sh, den_sh = scr[27], scr[28]
    isrc_sem = scr[29:35]
    idst_sem = scr[35:41]
    asv_sem = scr[41:44]
    adv_sem = scr[44:47]
    gat_sem = scr[47:49]
    den_sem = scr[49:55]
    scat_sem = scr[55:57]
    _sc_body(h_hbm, als_hbm, ald_hbm, src_hbm, dst_hbm,
             numer_hbm, den0_hbm, den1_hbm,
             den_v, src_v, dst_v, w_v, asv, adv, rows, acc_sh, den_sh,
             isrc_sem, idst_sem, asv_sem, adv_sem, gat_sem, den_sem,
             scat_sem)


def _combine(np_ref, d0_ref, d1_ref, h_ref, als_ref, ald_ref, b_ref):
    es = (als_ref[...] + ald_ref[...]).reshape(_N, 1)
    ws = jnp.exp(jnp.where(es >= 0, es, _SLOPE * es))
    numer = np_ref[0] + np_ref[1] + ws * h_ref[...]
    denom = (d0_ref[...] + d1_ref[...]).reshape(_N, 1) + ws
    return numer / (denom + 1e-16) + b_ref[...]


def _mid_body(np_ref, d0_ref, d1_ref, h_ref, als_ref, ald_ref, b_ref, w_ref,
              a2s_ref, a2d_ref, h2_ref, als2_ref, ald2_ref):
    r = jax.nn.relu(_combine(np_ref, d0_ref, d1_ref, h_ref, als_ref, ald_ref,
                             b_ref))
    h2 = jnp.dot(r, w_ref[...], preferred_element_type=jnp.float32)
    h2_ref[...] = h2
    als2_ref[...] = jnp.dot(h2, a2s_ref[...],
                            preferred_element_type=jnp.float32)[:, 0]
    ald2_ref[...] = jnp.dot(h2, a2d_ref[...],
                            preferred_element_type=jnp.float32)[:, 0]


def _fin_body(np_ref, d0_ref, d1_ref, h_ref, als_ref, ald_ref, b_ref, out_ref):
    o = _combine(np_ref, d0_ref, d1_ref, h_ref, als_ref, ald_ref, b_ref)
    m = jnp.max(o, axis=1, keepdims=True)
    lse = jnp.log(jnp.sum(jnp.exp(o - m), axis=1, keepdims=True)) + m
    out_ref[...] = o - lse


def _mid(np1, d0, d1, h, als, ald, b, W2, a2s, a2d):
    return pl.pallas_call(
        _mid_body,
        out_shape=[
            jax.ShapeDtypeStruct((_N, _D), jnp.float32),
            jax.ShapeDtypeStruct((_N,), jnp.float32),
            jax.ShapeDtypeStruct((_N,), jnp.float32),
        ],
    )(np1, d0, d1, h, als, ald, b, W2, a2s, a2d)


def _fin(np2, d0, d1, h, als, ald, b):
    return pl.pallas_call(
        _fin_body,
        out_shape=jax.ShapeDtypeStruct((_N, _D), jnp.float32),
    )(np2, d0, d1, h, als, ald, b)


def kernel(x, edge_index, W1, a_src1, a_dst1, b1, W2, a_src2, a_dst2, b2):
    pad = jnp.zeros((_BPW * _EB - _EPW,), jnp.int32)
    src = jnp.concatenate([edge_index[0], pad])
    dst = jnp.concatenate([edge_index[1], pad])
    h1, als1, ald1 = _dense(x, W1, a_src1.reshape(_D, 1), a_dst1.reshape(_D, 1))
    np1, dp10, dp11 = _sc_edges(h1, als1, ald1, src, dst)
    h2, als2, ald2 = _mid(np1, dp10, dp11, h1, als1, ald1, b1,
                          W2, a_src2.reshape(_D, 1), a_dst2.reshape(_D, 1))
    np2, dp20, dp21 = _sc_edges(h2, als2, ald2, src, dst)
    return _fin(np2, dp20, dp21, h2, als2, ald2, b2)
